# baseline (speedup 1.0000x reference)
import functools

import jax
import jax.numpy as jnp
import numpy as np
from jax import lax
from jax.experimental import pallas as pl
from jax.experimental.pallas import tpu as pltpu
from jax.experimental.pallas import tpu_sc as plsc

N_TOKENS = 1000000
EMBED_DIM = 64
BATCH = 4096
SEQ = 200

_NC = 2
_NS = 16
_NW = _NC * _NS
_BBLK = BATCH // _NW
_LANES = 16

_RC = 512


def _make_pos_enc():
    pos = np.arange(SEQ, dtype=np.float32)[:, None]
    i = np.arange(0, EMBED_DIM, 2, dtype=np.float32)[None, :]
    angle = pos / np.power(10000.0, i / float(EMBED_DIM))
    pe = np.zeros((SEQ, 128), dtype=np.float32)
    pe[:, 0:EMBED_DIM:2] = np.sin(angle)
    pe[:, 1:EMBED_DIM:2] = np.cos(angle)
    return jnp.asarray(pe)


def _tc_repack_kernel(tT_ref, out_ref):
    out_ref[:, 0:EMBED_DIM] = tT_ref[...].T


def _gather_kernel(t2_hbm, idxT_hbm, pe_hbm, out_hbm,
                   idx_v, rows_v, outT_v, pe_v, gsem, ssem):
    wid = lax.axis_index("s") * _NC + lax.axis_index("c")
    b0 = wid * _BBLK
    pltpu.sync_copy(pe_hbm, pe_v)
    pltpu.sync_copy(idxT_hbm.at[:, pl.ds(b0, _BBLK)], idx_v)
    iota = lax.iota(jnp.int32, _LANES)

    def issue(s, b):
        pltpu.async_copy(t2_hbm.at[idx_v.at[s]], rows_v.at[b], gsem.at[b])

    def wait_gather(s, b):
        pltpu.make_async_copy(
            t2_hbm.at[idx_v.at[s]], rows_v.at[b], gsem.at[b]).wait()

    def start_store(s, b):
        pltpu.async_copy(
            outT_v.at[b], out_hbm.at[s, :, pl.ds(b0, _BBLK)], ssem.at[b])

    def wait_store(s, b):
        pltpu.make_async_copy(
            outT_v.at[b], out_hbm.at[s, :, pl.ds(b0, _BBLK)],
            ssem.at[b]).wait()

    def process(s, b):
        for m in range(EMBED_DIM // _LANES):
            dvec = iota + m * _LANES
            pev = pe_v[s, pl.ds(m * _LANES, _LANES)]

            def rbody(r, carry):
                rot = jnp.bitwise_and(iota + r, _LANES - 1)
                for j0 in range(0, _BBLK, _LANES):
                    jvec = rot + j0
                    vals = plsc.load_gather(rows_v.at[b], [jvec, dvec])
                    plsc.store_scatter(outT_v.at[b], [dvec, jvec],
                                       vals + pev)
                return carry

            lax.fori_loop(0, _LANES, rbody, 0, unroll=4)

    issue(0, 0)

    def body(s, carry):
        b = lax.rem(s, 2)
        b1 = 1 - b

        @pl.when(s + 1 < SEQ)
        def _():
            issue(s + 1, b1)

        wait_gather(s, b)

        @pl.when(s >= 2)
        def _():
            wait_store(s - 2, b)

        process(s, b)
        start_store(s, b)
        return carry

    lax.fori_loop(0, SEQ, body, 0)
    wait_store(SEQ - 2, 0)
    wait_store(SEQ - 1, 1)


@jax.jit
def _gather_pe(inputs, table, pe):
    mesh = plsc.VectorSubcoreMesh(core_axis_name="c", subcore_axis_name="s")
    params = pltpu.CompilerParams(
        use_tc_tiling_on_sc=True, needs_layout_passes=False)

    nblk = (N_TOKENS + _RC - 1) // _RC
    t2 = pl.pallas_call(
        _tc_repack_kernel,
        grid=(nblk,),
        in_specs=[pl.BlockSpec((EMBED_DIM, _RC), lambda i: (0, i))],
        out_specs=pl.BlockSpec((_RC, 2 * EMBED_DIM), lambda i: (i, 0)),
        out_shape=jax.ShapeDtypeStruct((N_TOKENS, 2 * EMBED_DIM), jnp.float32),
    )(table.T)

    gather = functools.partial(
        pl.kernel,
        mesh=mesh,
        out_type=jax.ShapeDtypeStruct((SEQ, EMBED_DIM, BATCH), jnp.float32),
        scratch_types=[
            pltpu.VMEM((SEQ, _BBLK), jnp.int32),
            pltpu.VMEM((2, _BBLK, 2 * EMBED_DIM), jnp.float32),
            pltpu.VMEM((2, EMBED_DIM, _BBLK), jnp.float32),
            pltpu.VMEM((SEQ, 128), jnp.float32),
            pltpu.SemaphoreType.DMA((2,)),
            pltpu.SemaphoreType.DMA((2,)),
        ],
        compiler_params=params,
    )(_gather_kernel)
    out = gather(t2, inputs.T, pe)
    return jnp.transpose(out, (2, 0, 1))


def kernel(inputs, table):
    return _gather_pe(inputs, table, _make_pos_enc())


# speedup vs baseline: 1.9723x; 1.9723x over previous
import functools

import jax
import jax.numpy as jnp
import numpy as np
from jax import lax
from jax.experimental import pallas as pl
from jax.experimental.pallas import tpu as pltpu
from jax.experimental.pallas import tpu_sc as plsc

N_TOKENS = 1000000
EMBED_DIM = 64
BATCH = 4096
SEQ = 200

_NC = 2
_NS = 16
_NW = _NC * _NS
_BBLK = BATCH // _NW
_LANES = 16

_RC = 256
_NFULL = N_TOKENS // _RC
_RTAIL = N_TOKENS - _NFULL * _RC
_JMAX = (_NFULL + _NW - 1) // _NW


def _make_pos_enc():
    pos = np.arange(SEQ, dtype=np.float32)[:, None]
    i = np.arange(0, EMBED_DIM, 2, dtype=np.float32)[None, :]
    angle = pos / np.power(10000.0, i / float(EMBED_DIM))
    pe = np.zeros((SEQ, 128), dtype=np.float32)
    pe[:, 0:EMBED_DIM:2] = np.sin(angle)
    pe[:, 1:EMBED_DIM:2] = np.cos(angle)
    return jnp.asarray(pe)


def _transpose_block(src, dst, iota, ncols, nrows):
    def rbody(r, carry):
        rot = jnp.bitwise_and(iota + r, _LANES - 1)
        for m in range(EMBED_DIM // _LANES):
            dvec = iota + m * _LANES
            for c0 in range(0, ncols, _LANES):
                cvec = rot + c0
                vals = plsc.load_gather(src, [dvec, cvec])
                plsc.store_scatter(dst, [cvec, dvec], vals)
        return carry

    lax.fori_loop(0, _LANES, rbody, 0, unroll=4)


def _repack_kernel(tT_hbm, tail_hbm, t2_hbm, blk_v, out_v, lsem, ssem):
    wid = lax.axis_index("s") * _NC + lax.axis_index("c")
    iota = lax.iota(jnp.int32, _LANES)

    def blkid(j):
        return j * _NW + wid

    def start_load(j, b):
        pltpu.async_copy(
            tT_hbm.at[:, pl.ds(blkid(j) * _RC, _RC)], blk_v.at[b],
            lsem.at[b])

    def wait_load(j, b):
        pltpu.make_async_copy(
            tT_hbm.at[:, pl.ds(blkid(j) * _RC, _RC)], blk_v.at[b],
            lsem.at[b]).wait()

    def start_store(j, b):
        pltpu.async_copy(
            out_v.at[b], t2_hbm.at[pl.ds(blkid(j) * _RC, _RC), :],
            ssem.at[b])

    def wait_store(j, b):
        pltpu.make_async_copy(
            out_v.at[b], t2_hbm.at[pl.ds(blkid(j) * _RC, _RC), :],
            ssem.at[b]).wait()

    start_load(0, 0)

    def body(j, carry):
        b = lax.rem(j, 2)
        b1 = 1 - b

        @pl.when(blkid(j + 1) < _NFULL)
        def _():
            start_load(j + 1, b1)

        @pl.when(blkid(j) < _NFULL)
        def _():
            wait_load(j, b)

            @pl.when(j >= 2)
            def _():
                wait_store(j - 2, b)

            _transpose_block(blk_v.at[b], out_v.at[b], iota, _RC, _RC)
            start_store(j, b)

        return carry

    lax.fori_loop(0, _JMAX, body, 0)
    @pl.when(blkid(_JMAX - 1) < _NFULL)
    def _():
        wait_store(_JMAX - 2, (_JMAX - 2) % 2)
        wait_store(_JMAX - 1, (_JMAX - 1) % 2)

    @pl.when(blkid(_JMAX - 1) >= _NFULL)
    def _():
        wait_store(_JMAX - 3, (_JMAX - 3) % 2)
        wait_store(_JMAX - 2, (_JMAX - 2) % 2)

    @pl.when(wid == 0)
    def _():
        pltpu.sync_copy(tail_hbm, blk_v.at[0, :, pl.ds(0, 128)])
        _transpose_block(blk_v.at[0], out_v.at[0], iota, 128, _RC)
        pltpu.sync_copy(out_v.at[0, pl.ds(0, 64), :],
                        t2_hbm.at[pl.ds(_NFULL * _RC, _RTAIL), :])


def _gather_kernel(t2_hbm, idxT_hbm, pe_hbm, out_hbm,
                   idx_v, rows_v, outT_v, pe_v, gsem, ssem):
    wid = lax.axis_index("s") * _NC + lax.axis_index("c")
    b0 = wid * _BBLK
    pltpu.sync_copy(pe_hbm, pe_v)
    pltpu.sync_copy(idxT_hbm.at[:, pl.ds(b0, _BBLK)], idx_v)
    iota = lax.iota(jnp.int32, _LANES)

    def issue(s, b):
        pltpu.async_copy(t2_hbm.at[idx_v.at[s]], rows_v.at[b], gsem.at[b])

    def wait_gather(s, b):
        pltpu.make_async_copy(
            t2_hbm.at[idx_v.at[s]], rows_v.at[b], gsem.at[b]).wait()

    def start_store(s, b):
        pltpu.async_copy(
            outT_v.at[b], out_hbm.at[s, :, pl.ds(b0, _BBLK)], ssem.at[b])

    def wait_store(s, b):
        pltpu.make_async_copy(
            outT_v.at[b], out_hbm.at[s, :, pl.ds(b0, _BBLK)],
            ssem.at[b]).wait()

    def process(s, b):
        for m in range(EMBED_DIM // _LANES):
            dvec = iota + m * _LANES
            pev = pe_v[s, pl.ds(m * _LANES, _LANES)]

            def rbody(r, carry):
                rot = jnp.bitwise_and(iota + r, _LANES - 1)
                for j0 in range(0, _BBLK, _LANES):
                    jvec = rot + j0
                    vals = plsc.load_gather(rows_v.at[b], [jvec, dvec])
                    plsc.store_scatter(outT_v.at[b], [dvec, jvec],
                                       vals + pev)
                return carry

            lax.fori_loop(0, _LANES, rbody, 0, unroll=8)

    issue(0, 0)

    def body(s, carry):
        b = lax.rem(s, 2)
        b1 = 1 - b

        @pl.when(s + 1 < SEQ)
        def _():
            issue(s + 1, b1)

        wait_gather(s, b)

        @pl.when(s >= 2)
        def _():
            wait_store(s - 2, b)

        process(s, b)
        start_store(s, b)
        return carry

    lax.fori_loop(0, SEQ, body, 0)
    wait_store(SEQ - 2, 0)
    wait_store(SEQ - 1, 1)


@jax.jit
def _gather_pe(inputs, table, pe):
    mesh = plsc.VectorSubcoreMesh(core_axis_name="c", subcore_axis_name="s")
    params = pltpu.CompilerParams(
        use_tc_tiling_on_sc=True, needs_layout_passes=False)

    repack = functools.partial(
        pl.kernel,
        mesh=mesh,
        out_type=jax.ShapeDtypeStruct((N_TOKENS, 2 * EMBED_DIM), jnp.float32),
        scratch_types=[
            pltpu.VMEM((2, EMBED_DIM, _RC), jnp.float32),
            pltpu.VMEM((2, _RC, 2 * EMBED_DIM), jnp.float32),
            pltpu.SemaphoreType.DMA((2,)),
            pltpu.SemaphoreType.DMA((2,)),
        ],
        compiler_params=params,
    )(_repack_kernel)
    tail = jnp.pad(lax.slice(table, (_NFULL * _RC, 0), (N_TOKENS, EMBED_DIM)),
                   ((0, 128 - _RTAIL), (0, 0))).T
    t2 = repack(table.T, tail)

    gather = functools.partial(
        pl.kernel,
        mesh=mesh,
        out_type=jax.ShapeDtypeStruct((SEQ, EMBED_DIM, BATCH), jnp.float32),
        scratch_types=[
            pltpu.VMEM((SEQ, _BBLK), jnp.int32),
            pltpu.VMEM((2, _BBLK, 2 * EMBED_DIM), jnp.float32),
            pltpu.VMEM((2, EMBED_DIM, _BBLK), jnp.float32),
            pltpu.VMEM((SEQ, 128), jnp.float32),
            pltpu.SemaphoreType.DMA((2,)),
            pltpu.SemaphoreType.DMA((2,)),
        ],
        compiler_params=params,
    )(_gather_kernel)
    out = gather(t2, inputs.T, pe)
    return jnp.transpose(out, (2, 0, 1))


def kernel(inputs, table):
    return _gather_pe(inputs, table, _make_pos_enc())
